# Initial kernel scaffold; baseline (speedup 1.0000x reference)
import jax
import jax.numpy as jnp
from jax.experimental import pallas as pl


def kernel(idx, targets, table):
    raise NotImplementedError("write your pallas kernel here")


# speedup vs baseline: 1.4529x; 1.4529x over previous
import functools

import jax
import jax.numpy as jnp
from jax import lax
from jax.experimental import pallas as pl
from jax.experimental.pallas import tpu as pltpu, tpu_sc as plsc

VOCAB = 1000


def _rowlz_body(t_ref, o_ref):
    t = t_ref[...]
    m = jnp.max(t, axis=1)
    s = jnp.sum(jnp.exp(t - m[:, None]), axis=1)
    o_ref[...] = m + jnp.log(s)


def _rowlz(table):
    return pl.pallas_call(
        _rowlz_body,
        out_shape=jax.ShapeDtypeStruct((VOCAB,), jnp.float32),
    )(table)


def _make_sc_kernel(N, D, NC, NS, L):
    NW = NC * NS
    b_per_w = N // NW
    CH = 64
    n_ch = b_per_w // CH
    mesh = plsc.VectorSubcoreMesh(core_axis_name="c", subcore_axis_name="s")

    @functools.partial(
        pl.kernel,
        out_type=(
            jax.ShapeDtypeStruct((N, D), jnp.float32),
            jax.ShapeDtypeStruct((NW, L), jnp.float32),
        ),
        mesh=mesh,
        scratch_types=[
            pltpu.VMEM((b_per_w,), jnp.int32),
            pltpu.VMEM((b_per_w,), jnp.int32),
            pltpu.VMEM((VOCAB,), jnp.float32),
            pltpu.VMEM((CH, D), jnp.float32),
            pltpu.VMEM((L,), jnp.float32),
            pltpu.SemaphoreType.DMA,
        ],
        compiler_params=pltpu.CompilerParams(use_tc_tiling_on_sc=False,
                                             needs_layout_passes=False),
    )
    def sc_kernel(table_hbm, idx_hbm, tgt_hbm, lz_hbm,
                  out_hbm, part_hbm,
                  idx_v, tgt_v, lz_v, rows_v, acc_v, sem):
        wid = lax.axis_index("s") * NC + lax.axis_index("c")
        base = wid * b_per_w
        pltpu.sync_copy(idx_hbm.at[pl.ds(base, b_per_w)], idx_v)
        pltpu.sync_copy(tgt_hbm.at[pl.ds(base, b_per_w)], tgt_v)
        pltpu.sync_copy(lz_hbm, lz_v)
        lane = lax.iota(jnp.int32, L)

        def chunk(c, acc):
            off = c * CH
            pltpu.async_copy(table_hbm.at[idx_v.at[pl.ds(off, CH)]],
                             rows_v, sem).wait()
            pltpu.sync_copy(rows_v, out_hbm.at[pl.ds(base + off, CH)])
            for j in range(CH // L):
                idx16 = idx_v[pl.ds(off + j * L, L)]
                tg16 = tgt_v[pl.ds(off + j * L, L)]
                lg = plsc.load_gather(lz_v, [idx16])
                pk = plsc.load_gather(rows_v, [j * L + lane, tg16])
                acc = acc + (lg - pk)
            return acc

        acc = lax.fori_loop(0, n_ch, chunk, jnp.zeros((L,), jnp.float32))
        acc_v[...] = acc
        pltpu.sync_copy(acc_v, part_hbm.at[wid])

    return sc_kernel


def kernel(idx, targets, table):
    B, T = idx.shape
    V, D = table.shape
    N = B * T
    info = plsc.get_sparse_core_info()
    NC, NS, L = info.num_cores, info.num_subcores, info.num_lanes
    idx_f = idx.reshape(N).astype(jnp.int32)
    tgt_f = targets.reshape(N).astype(jnp.int32)
    lz = _rowlz(table)
    sc = _make_sc_kernel(N, D, NC, NS, L)
    logits, partials = sc(table, idx_f, tgt_f, lz)
    loss = jnp.sum(partials) / jnp.float32(N)
    return (logits, loss)


# speedup vs baseline: 2.1827x; 1.5022x over previous
import functools

import jax
import jax.numpy as jnp
from jax import lax
from jax.experimental import pallas as pl
from jax.experimental.pallas import tpu as pltpu, tpu_sc as plsc

VOCAB = 1000
DPAD = 1024


def _rowlz_body(t_ref, o_ref):
    t = t_ref[...]
    m = jnp.max(t, axis=1)
    s = jnp.sum(jnp.exp(t - m[:, None]), axis=1)
    lz = m + jnp.log(s)
    o_ref[...] = jnp.concatenate(
        [lz, jnp.zeros((DPAD - VOCAB,), jnp.float32)]).reshape(8, 128)


def _rowlz(table):
    return pl.pallas_call(
        _rowlz_body,
        out_shape=jax.ShapeDtypeStruct((8, 128), jnp.float32),
    )(table)


def _make_sc_kernel(N, D, NC, NS, L):
    NW = NC * NS
    CH = 64
    n_ch = N // CH // NW
    LT = DPAD // 128
    mesh = plsc.VectorSubcoreMesh(core_axis_name="c", subcore_axis_name="s")

    @functools.partial(
        pl.kernel,
        out_type=(
            jax.ShapeDtypeStruct((N, DPAD), jnp.float32),
            jax.ShapeDtypeStruct((NW, 128), jnp.float32),
        ),
        mesh=mesh,
        scratch_types=[
            pltpu.VMEM((CH,), jnp.int32),
            pltpu.VMEM((CH,), jnp.int32),
            pltpu.VMEM((8, 128), jnp.float32),
            pltpu.VMEM((CH, 8, 128), jnp.float32),
            pltpu.VMEM((128,), jnp.float32),
            pltpu.SemaphoreType.DMA,
        ],
        compiler_params=pltpu.CompilerParams(use_tc_tiling_on_sc=True,
                                             needs_layout_passes=False),
    )
    def sc_kernel(table_hbm, idx_hbm, tgt_hbm, lz_hbm,
                  out_hbm, part_hbm,
                  idx_v, tgt_v, lz_v, rows_v, acc_v, sem):
        wid = lax.axis_index("s") * NC + lax.axis_index("c")
        pltpu.sync_copy(lz_hbm, lz_v)
        lane = lax.iota(jnp.int32, L)

        def chunk(k, acc):
            c = wid + k * NW
            tok0 = c * CH
            pltpu.sync_copy(idx_hbm.at[c], idx_v)
            pltpu.sync_copy(tgt_hbm.at[c], tgt_v)
            pltpu.async_copy(table_hbm.at[idx_v], rows_v, sem).wait()
            for l in range(LT):
                pltpu.sync_copy(
                    rows_v.at[:, l],
                    out_hbm.at[pl.ds(tok0, CH), pl.ds(l * 128, 128)])
            for j in range(CH // L):
                idx16 = idx_v[pl.ds(j * L, L)]
                tg16 = tgt_v[pl.ds(j * L, L)]
                lg = plsc.load_gather(lz_v, [idx16 // 128, idx16 % 128])
                pk = plsc.load_gather(
                    rows_v, [j * L + lane, tg16 // 128, tg16 % 128])
                acc = acc + (lg - pk)
            return acc

        acc = lax.fori_loop(0, n_ch, chunk, jnp.zeros((L,), jnp.float32))
        acc_v[pl.ds(0, L)] = acc
        acc_v[pl.ds(L, L)] = jnp.zeros((L,), jnp.float32)
        for j in range(2, 128 // L):
            acc_v[pl.ds(j * L, L)] = jnp.zeros((L,), jnp.float32)
        pltpu.sync_copy(acc_v, part_hbm.at[wid])

    return sc_kernel


def kernel(idx, targets, table):
    B, T = idx.shape
    V, D = table.shape
    N = B * T
    info = plsc.get_sparse_core_info()
    NC, NS, L = info.num_cores, info.num_subcores, info.num_lanes
    idx2 = idx.reshape(N // 64, 64).astype(jnp.int32)
    tgt2 = targets.reshape(N // 64, 64).astype(jnp.int32)
    table3 = jnp.pad(table, ((0, 0), (0, DPAD - D))).reshape(V, 8, 128)
    lz = _rowlz(table)
    sc = _make_sc_kernel(N, D, NC, NS, L)
    logits_pad, partials = sc(table3, idx2, tgt2, lz)
    logits = logits_pad[:, :D]
    loss = jnp.sum(partials) / jnp.float32(N)
    return (logits, loss)


# speedup vs baseline: 2.4251x; 1.1111x over previous
import functools

import jax
import jax.numpy as jnp
from jax import lax
from jax.experimental import pallas as pl
from jax.experimental.pallas import tpu as pltpu, tpu_sc as plsc

VOCAB = 1000
DPAD = 1024


def _rowlz_body(t_ref, o_ref):
    t = t_ref[...]
    m = jnp.max(t, axis=1)
    s = jnp.sum(jnp.exp(t - m[:, None]), axis=1)
    lz = m + jnp.log(s)
    o_ref[...] = jnp.concatenate(
        [lz, jnp.zeros((DPAD - VOCAB,), jnp.float32)]).reshape(8, 128)


def _rowlz(table):
    return pl.pallas_call(
        _rowlz_body,
        out_shape=jax.ShapeDtypeStruct((8, 128), jnp.float32),
    )(table)


def _make_sc_kernel(N, D, NC, NS, L):
    NW = NC * NS
    CH = 32
    n_ch = N // CH // NW
    LT = DPAD // 128
    mesh = plsc.VectorSubcoreMesh(core_axis_name="c", subcore_axis_name="s")

    @functools.partial(
        pl.kernel,
        out_type=(
            jax.ShapeDtypeStruct((N, DPAD), jnp.float32),
            jax.ShapeDtypeStruct((NW, 128), jnp.float32),
        ),
        mesh=mesh,
        scratch_types=[
            pltpu.VMEM((n_ch, CH), jnp.int32),
            pltpu.VMEM((n_ch, CH), jnp.int32),
            pltpu.VMEM((8, 128), jnp.float32),
            pltpu.VMEM((CH, 8, 128), jnp.float32),
            pltpu.VMEM((CH, 8, 128), jnp.float32),
            pltpu.VMEM((128,), jnp.float32),
            pltpu.SemaphoreType.DMA,
            pltpu.SemaphoreType.DMA,
            pltpu.SemaphoreType.DMA,
            pltpu.SemaphoreType.DMA,
        ],
        compiler_params=pltpu.CompilerParams(use_tc_tiling_on_sc=True,
                                             needs_layout_passes=False),
    )
    def sc_kernel(table_hbm, idx_hbm, tgt_hbm, lz_hbm,
                  out_hbm, part_hbm,
                  idx_v, tgt_v, lz_v, rows_a, rows_b, acc_v,
                  gsem_a, gsem_b, ssem_a, ssem_b):
        wid = lax.axis_index("s") * NC + lax.axis_index("c")
        pltpu.sync_copy(idx_hbm.at[:, wid], idx_v)
        pltpu.sync_copy(tgt_hbm.at[:, wid], tgt_v)
        pltpu.sync_copy(lz_hbm, lz_v)
        lane = lax.iota(jnp.int32, L)

        def tok0_of(k):
            return (k * NW + wid) * CH

        def start_gather(k, buf, sem):
            return pltpu.async_copy(table_hbm.at[idx_v.at[k]], buf, sem)

        def issue_stores(k, rows, sem):
            tok0 = tok0_of(k)
            return [pltpu.async_copy(
                rows.at[:, l],
                out_hbm.at[pl.ds(tok0, CH), pl.ds(l * 128, 128)],
                sem) for l in range(LT)]

        def loss(k, rows, acc):
            for j in range(CH // L):
                idx16 = idx_v[k, pl.ds(j * L, L)]
                tg16 = tgt_v[k, pl.ds(j * L, L)]
                lg = plsc.load_gather(lz_v, [idx16 // 128, idx16 % 128])
                pk = plsc.load_gather(
                    rows, [j * L + lane, tg16 // 128, tg16 % 128])
                acc = acc + (lg - pk)
            return acc

        n_half = n_ch // 2
        start_gather(0, rows_a, gsem_a)

        def body(g, acc):
            k0 = 2 * g
            pltpu.make_async_copy(
                table_hbm.at[idx_v.at[k0]], rows_a, gsem_a).wait()
            hb = start_gather(k0 + 1, rows_b, gsem_b)
            sha = issue_stores(k0, rows_a, ssem_a)
            acc = loss(k0, rows_a, acc)
            hb.wait()
            for h in sha:
                h.wait()

            @pl.when(g + 1 < n_half)
            def _():
                start_gather(k0 + 2, rows_a, gsem_a)

            shb = issue_stores(k0 + 1, rows_b, ssem_b)
            acc = loss(k0 + 1, rows_b, acc)
            for h in shb:
                h.wait()
            return acc

        acc = lax.fori_loop(0, n_half, body, jnp.zeros((L,), jnp.float32))

        acc_v[pl.ds(0, L)] = acc
        for j in range(1, 128 // L):
            acc_v[pl.ds(j * L, L)] = jnp.zeros((L,), jnp.float32)
        pltpu.sync_copy(acc_v, part_hbm.at[wid])

    return sc_kernel


def kernel(idx, targets, table):
    B, T = idx.shape
    V, D = table.shape
    N = B * T
    info = plsc.get_sparse_core_info()
    NC, NS, L = info.num_cores, info.num_subcores, info.num_lanes
    NW = NC * NS
    idx3 = idx.reshape(N // (32 * NW), NW, 32).astype(jnp.int32)
    tgt3 = targets.reshape(N // (32 * NW), NW, 32).astype(jnp.int32)
    table3 = jnp.pad(table, ((0, 0), (0, DPAD - D))).reshape(V, 8, 128)
    lz = _rowlz(table)
    sc = _make_sc_kernel(N, D, NC, NS, L)
    logits_pad, partials = sc(table3, idx3, tgt3, lz)
    logits = logits_pad[:, :D]
    loss = jnp.sum(partials) / jnp.float32(N)
    return (logits, loss)
